# Initial kernel scaffold; baseline (speedup 1.0000x reference)
import jax
import jax.numpy as jnp
from jax.experimental import pallas as pl


def kernel(centers, quaternions, scales, density):
    raise NotImplementedError("write your pallas kernel here")


# speedup vs baseline: 24.9299x; 24.9299x over previous
import functools

import jax
import jax.numpy as jnp
from jax import lax
from jax.experimental import pallas as pl
from jax.experimental.pallas import tpu as pltpu

D = H = W = 128
N = 8192
WIN = 10
YW = 24


def _prep_body(cen_ref, quat_ref, sc_ref, den_ref, f_ref, i_ref):
    cz = cen_ref[0:1, :] * (D - 1.0)
    cy = cen_ref[1:2, :] * (H - 1.0)
    cx = cen_ref[2:3, :] * (W - 1.0)

    qw = quat_ref[0:1, :]
    qx = quat_ref[1:2, :]
    qy = quat_ref[2:3, :]
    qz = quat_ref[3:4, :]
    norm = jnp.sqrt(qw * qw + qx * qx + qy * qy + qz * qz) + 1e-08
    w = qw / norm
    x = qx / norm
    y = qy / norm
    z = qz / norm

    r00 = 1 - 2 * (y * y + z * z)
    r01 = 2 * (x * y - z * w)
    r02 = 2 * (x * z + y * w)
    r10 = 2 * (x * y + z * w)
    r11 = 1 - 2 * (x * x + z * z)
    r12 = 2 * (y * z - x * w)
    r20 = 2 * (x * z - y * w)
    r21 = 2 * (y * z + x * w)
    r22 = 1 - 2 * (x * x + y * y)

    s0 = 0.008 + sc_ref[0:1, :] * 0.015
    s1 = 0.008 + sc_ref[1:2, :] * 0.015
    s2 = 0.008 + sc_ref[2:3, :] * 0.015
    i0 = 1.0 / (s0 * s0)
    i1 = 1.0 / (s1 * s1)
    i2 = 1.0 / (s2 * s2)

    sc = 1.0 / 4096.0
    caa = (r00 * r00 * i0 + r01 * r01 * i1 + r02 * r02 * i2) * sc
    cbb = (r10 * r10 * i0 + r11 * r11 * i1 + r12 * r12 * i2) * sc
    ccc = (r20 * r20 * i0 + r21 * r21 * i1 + r22 * r22 * i2) * sc
    cab = (r00 * r10 * i0 + r01 * r11 * i1 + r02 * r12 * i2) * (2.0 * sc)
    cac = (r00 * r20 * i0 + r01 * r21 * i1 + r02 * r22 * i2) * (2.0 * sc)
    cbc = (r10 * r20 * i0 + r11 * r21 * i1 + r12 * r22 * i2) * (2.0 * sc)

    f_ref[0:1, :] = caa
    f_ref[1:2, :] = cbb
    f_ref[2:3, :] = ccc
    f_ref[3:4, :] = cab
    f_ref[4:5, :] = cac
    f_ref[5:6, :] = cbc
    f_ref[6:7, :] = cz
    f_ref[7:8, :] = cy
    f_ref[8:9, :] = cx
    f_ref[9:10, :] = den_ref[0:1, :]

    bz = jnp.floor(cz).astype(jnp.int32)
    by = jnp.floor(cy).astype(jnp.int32)
    z0 = jnp.clip(bz - 4, 0, D - WIN)
    y0 = jnp.clip(jnp.bitwise_and(by - 4, -8), 0, H - YW)
    i_ref[0:1, :] = z0
    i_ref[1:2, :] = y0


def _vox_body(i_ref, f_ref, out_ref):
    n = pl.program_id(0)

    @pl.when(n == 0)
    def _():
        out_ref[...] = jnp.zeros((D, H, W), jnp.float32)

    z0 = i_ref[0, n]
    y0 = i_ref[1, n]
    czz = f_ref[0, n]
    cyy = f_ref[1, n]
    cxx = f_ref[2, n]
    czy = f_ref[3, n]
    czx = f_ref[4, n]
    cyx = f_ref[5, n]
    cz = f_ref[6, n]
    cy = f_ref[7, n]
    cx = f_ref[8, n]
    dens = f_ref[9, n]

    zi = lax.broadcasted_iota(jnp.int32, (WIN, YW, W), 0).astype(jnp.float32)
    yi = lax.broadcasted_iota(jnp.int32, (WIN, YW, W), 1).astype(jnp.float32)
    xi = lax.broadcasted_iota(jnp.int32, (WIN, YW, W), 2).astype(jnp.float32)
    dz = zi + (z0.astype(jnp.float32) - cz)
    dy = yi + (y0.astype(jnp.float32) - cy)
    dx = xi - cx

    m = (czz * dz * dz + cyy * dy * dy + cxx * dx * dx
         + czy * dz * dy + czx * dz * dx + cyx * dy * dx)
    w = jnp.exp(-0.5 * m) * dens
    w = jnp.where(m <= 9.0, w, 0.0)

    out_ref[pl.ds(z0, WIN), pl.ds(y0, YW), :] += w


@jax.jit
def kernel(centers, quaternions, scales, density):
    cen_t = centers.T.reshape(3, N)
    quat_t = quaternions.T.reshape(4, N)
    sc_t = scales.T.reshape(3, N)
    den_t = density.reshape(1, N)

    fparams, iparams = pl.pallas_call(
        _prep_body,
        out_shape=[
            jax.ShapeDtypeStruct((10, N), jnp.float32),
            jax.ShapeDtypeStruct((2, N), jnp.int32),
        ],
    )(cen_t, quat_t, sc_t, den_t)

    grid_spec = pltpu.PrefetchScalarGridSpec(
        num_scalar_prefetch=2,
        grid=(N,),
        in_specs=[],
        out_specs=pl.BlockSpec((D, H, W), lambda n, i_ref, f_ref: (0, 0, 0)),
    )
    volume = pl.pallas_call(
        _vox_body,
        grid_spec=grid_spec,
        out_shape=jax.ShapeDtypeStruct((D, H, W), jnp.float32),
        compiler_params=pltpu.CompilerParams(
            dimension_semantics=("arbitrary",),
        ),
    )(iparams, fparams)
    return volume


# speedup vs baseline: 86.5373x; 3.4712x over previous
import functools

import jax
import jax.numpy as jnp
from jax import lax
from jax.experimental import pallas as pl
from jax.experimental.pallas import tpu as pltpu
from jax.experimental.pallas import tpu_sc as plsc

D = H = W = 128
N = 8192
WIN = 10
NTILE = 32
SLABZ = D // NTILE
SLABW = SLABZ * H * W
NVREG = 7


def _prep_body(cen_ref, quat_ref, sc_ref, den_ref, f_ref, i_ref):
    cz = cen_ref[0:1, :] * (D - 1.0)
    cy = cen_ref[1:2, :] * (H - 1.0)
    cx = cen_ref[2:3, :] * (W - 1.0)

    qw = quat_ref[0:1, :]
    qx = quat_ref[1:2, :]
    qy = quat_ref[2:3, :]
    qz = quat_ref[3:4, :]
    norm = jnp.sqrt(qw * qw + qx * qx + qy * qy + qz * qz) + 1e-08
    w = qw / norm
    x = qx / norm
    y = qy / norm
    z = qz / norm

    r00 = 1 - 2 * (y * y + z * z)
    r01 = 2 * (x * y - z * w)
    r02 = 2 * (x * z + y * w)
    r10 = 2 * (x * y + z * w)
    r11 = 1 - 2 * (x * x + z * z)
    r12 = 2 * (y * z - x * w)
    r20 = 2 * (x * z - y * w)
    r21 = 2 * (y * z + x * w)
    r22 = 1 - 2 * (x * x + y * y)

    s0 = 0.008 + sc_ref[0:1, :] * 0.015
    s1 = 0.008 + sc_ref[1:2, :] * 0.015
    s2 = 0.008 + sc_ref[2:3, :] * 0.015
    i0 = 1.0 / (s0 * s0)
    i1 = 1.0 / (s1 * s1)
    i2 = 1.0 / (s2 * s2)

    sc = 1.0 / 4096.0
    f_ref[0:1, :] = (r00 * r00 * i0 + r01 * r01 * i1 + r02 * r02 * i2) * sc
    f_ref[1:2, :] = (r10 * r10 * i0 + r11 * r11 * i1 + r12 * r12 * i2) * sc
    f_ref[2:3, :] = (r20 * r20 * i0 + r21 * r21 * i1 + r22 * r22 * i2) * sc
    f_ref[3:4, :] = (r00 * r10 * i0 + r01 * r11 * i1 + r02 * r12 * i2) * (2.0 * sc)
    f_ref[4:5, :] = (r00 * r20 * i0 + r01 * r21 * i1 + r02 * r22 * i2) * (2.0 * sc)
    f_ref[5:6, :] = (r10 * r20 * i0 + r11 * r21 * i1 + r12 * r22 * i2) * (2.0 * sc)
    f_ref[6:7, :] = cz
    f_ref[7:8, :] = cy
    f_ref[8:9, :] = cx
    f_ref[9:10, :] = den_ref[0:1, :]

    z0 = jnp.clip(jnp.floor(cz).astype(jnp.int32) - 4, 0, D - WIN)
    y0 = jnp.clip(jnp.floor(cy).astype(jnp.int32) - 4, 0, H - WIN)
    x0 = jnp.clip(jnp.floor(cx).astype(jnp.int32) - 4, 0, W - WIN)
    f_ref[10:11, :] = lax.bitcast_convert_type(z0, jnp.float32)
    f_ref[11:12, :] = lax.bitcast_convert_type(y0, jnp.float32)
    f_ref[12:13, :] = lax.bitcast_convert_type(x0, jnp.float32)
    zero = cz * 0.0
    f_ref[13:14, :] = zero
    f_ref[14:15, :] = zero
    f_ref[15:16, :] = zero
    i_ref[0:1, :] = z0


def _sc_body(params_hbm, z0_hbm, out_hbm, vol_v, zv, wl, pstage, sem):
    i32 = jnp.int32
    f32 = jnp.float32
    wid = lax.axis_index("s") * 2 + lax.axis_index("c")
    lo = wid * SLABZ

    yoffs, xoffs, idxcs, padms = [], [], [], []
    for v in range(NVREG):
        p = lax.iota(i32, 16) + (16 * v)
        j = p // 10
        l = p % 10
        padm = p < 100
        yoffs.append(j.astype(f32))
        xoffs.append(l.astype(f32))
        idxcs.append(jnp.where(padm, j * W + l, 0))
        padms.append(padm)

    zero16 = jnp.zeros((16,), f32)

    def zbody(i, c):
        vol_v[pl.ds(i * 16, 16)] = zero16
        return c

    lax.fori_loop(0, SLABW // 16, zbody, 0)

    pltpu.sync_copy(z0_hbm, zv)

    def rbody(i, cnt):
        z0v = zv[pl.ds(i * 16, 16)]
        m = (z0v >= lo - (WIN - 1)) & (z0v <= lo + (SLABZ - 1))
        cs = plsc.cumsum(m.astype(i32))
        posv = cs + (cnt - 1)
        plsc.store_scatter(wl, [posv], lax.iota(i32, 16) + i * 16, mask=m)
        return cnt + jnp.max(cs)

    cnt = lax.fori_loop(0, N // 16, rbody, 0)
    plsc.store_scatter(wl, [lax.iota(i32, 16) + cnt], jnp.full((16,), N, i32))
    nch = (cnt + 15) // 16

    def cbody(ci, c):
        gidv = wl[pl.ds(ci * 16, 16)]
        pltpu.async_copy(params_hbm.at[gidv], pstage, sem).wait()

        def gbody(g, c2):
            row = pstage[g, :]

            def sp(k):
                return row.at[jnp.full((16,), k, i32)].get(
                    mode="promise_in_bounds")

            caa, cbb, ccc = sp(0), sp(1), sp(2)
            cab, cac, cbc = sp(3), sp(4), sp(5)
            czc, cyc, cxc = sp(6), sp(7), sp(8)
            dens = sp(9)
            z0iv = plsc.bitcast(sp(10), i32)
            y0iv = plsc.bitcast(sp(11), i32)
            x0iv = plsc.bitcast(sp(12), i32)
            z0s = jnp.max(z0iv)
            y0s = jnp.max(y0iv)
            x0s = jnp.max(x0iv)
            ybase = y0iv.astype(f32) - cyc
            xbase = x0iv.astype(f32) - cxc
            zlo = jnp.maximum(z0s, lo)
            zhi = jnp.minimum(z0s + WIN, lo + SLABZ)
            fb = y0s * W + x0s

            def pbody(z, c3):
                dz = jnp.broadcast_to(z, (16,)).astype(f32) - czc
                zq = caa * dz * dz
                zy = cab * dz
                zx = cac * dz
                pb = (z - lo) * (H * W) + fb
                for v in range(NVREG):
                    dy = ybase + yoffs[v]
                    dx = xbase + xoffs[v]
                    m = (zq + cbb * dy * dy + ccc * dx * dx
                         + zy * dy + zx * dx + cbc * dy * dx)
                    wv = jnp.exp(-0.5 * m) * dens
                    wv = jnp.where(m <= 9.0, wv, 0.0)
                    plsc.addupdate_scatter(vol_v, [idxcs[v] + pb], wv,
                                           mask=padms[v])
                return c3

            lax.fori_loop(zlo, zhi, pbody, 0)
            return c2

        lax.fori_loop(0, 16, gbody, 0)
        return c

    lax.fori_loop(0, nch, cbody, 0)

    pltpu.sync_copy(vol_v, out_hbm.at[pl.ds(wid * SLABW, SLABW)])


@jax.jit
def kernel(centers, quaternions, scales, density):
    cen_t = centers.T.reshape(3, N)
    quat_t = quaternions.T.reshape(4, N)
    sc_t = scales.T.reshape(3, N)
    den_t = density.reshape(1, N)

    fparams, iparams = pl.pallas_call(
        _prep_body,
        out_shape=[
            jax.ShapeDtypeStruct((16, N), jnp.float32),
            jax.ShapeDtypeStruct((1, N), jnp.int32),
        ],
    )(cen_t, quat_t, sc_t, den_t)

    params_nt = jnp.concatenate(
        [fparams.T, jnp.zeros((16, 16), jnp.float32)], axis=0)
    z0r = iparams.reshape(N)

    mesh = plsc.VectorSubcoreMesh(core_axis_name="c", subcore_axis_name="s")
    volume_flat = pl.kernel(
        _sc_body,
        out_type=jax.ShapeDtypeStruct((D * H * W,), jnp.float32),
        mesh=mesh,
        scratch_types=[
            pltpu.VMEM((SLABW,), jnp.float32),
            pltpu.VMEM((N,), jnp.int32),
            pltpu.VMEM((N + 16,), jnp.int32),
            pltpu.VMEM((16, 16), jnp.float32),
            pltpu.SemaphoreType.DMA,
        ],
        compiler_params=pltpu.CompilerParams(
            needs_layout_passes=False, use_tc_tiling_on_sc=False),
    )(params_nt, z0r)
    return volume_flat.reshape(D, H, W)


# speedup vs baseline: 106.3040x; 1.2284x over previous
import functools

import jax
import jax.numpy as jnp
from jax import lax
from jax.experimental import pallas as pl
from jax.experimental.pallas import tpu as pltpu
from jax.experimental.pallas import tpu_sc as plsc

D = H = W = 128
N = 8192
WIN = 10
NTILE = 32
SLABZ = D // NTILE
SLABW = SLABZ * H * W
NVREG = 7


def _prep_body(cen_ref, quat_ref, sc_ref, den_ref, f_ref, i_ref):
    cz = cen_ref[0:1, :] * (D - 1.0)
    cy = cen_ref[1:2, :] * (H - 1.0)
    cx = cen_ref[2:3, :] * (W - 1.0)

    qw = quat_ref[0:1, :]
    qx = quat_ref[1:2, :]
    qy = quat_ref[2:3, :]
    qz = quat_ref[3:4, :]
    norm = jnp.sqrt(qw * qw + qx * qx + qy * qy + qz * qz) + 1e-08
    w = qw / norm
    x = qx / norm
    y = qy / norm
    z = qz / norm

    r00 = 1 - 2 * (y * y + z * z)
    r01 = 2 * (x * y - z * w)
    r02 = 2 * (x * z + y * w)
    r10 = 2 * (x * y + z * w)
    r11 = 1 - 2 * (x * x + z * z)
    r12 = 2 * (y * z - x * w)
    r20 = 2 * (x * z - y * w)
    r21 = 2 * (y * z + x * w)
    r22 = 1 - 2 * (x * x + y * y)

    s0 = 0.008 + sc_ref[0:1, :] * 0.015
    s1 = 0.008 + sc_ref[1:2, :] * 0.015
    s2 = 0.008 + sc_ref[2:3, :] * 0.015
    i0 = 1.0 / (s0 * s0)
    i1 = 1.0 / (s1 * s1)
    i2 = 1.0 / (s2 * s2)

    sc = 1.0 / 4096.0
    f_ref[0:1, :] = (r00 * r00 * i0 + r01 * r01 * i1 + r02 * r02 * i2) * sc
    f_ref[1:2, :] = (r10 * r10 * i0 + r11 * r11 * i1 + r12 * r12 * i2) * sc
    f_ref[2:3, :] = (r20 * r20 * i0 + r21 * r21 * i1 + r22 * r22 * i2) * sc
    f_ref[3:4, :] = (r00 * r10 * i0 + r01 * r11 * i1 + r02 * r12 * i2) * (2.0 * sc)
    f_ref[4:5, :] = (r00 * r20 * i0 + r01 * r21 * i1 + r02 * r22 * i2) * (2.0 * sc)
    f_ref[5:6, :] = (r10 * r20 * i0 + r11 * r21 * i1 + r12 * r22 * i2) * (2.0 * sc)
    f_ref[6:7, :] = cz
    f_ref[7:8, :] = cy
    f_ref[8:9, :] = cx
    f_ref[9:10, :] = den_ref[0:1, :]

    z0 = jnp.clip(jnp.floor(cz).astype(jnp.int32) - 4, 0, D - WIN)
    y0 = jnp.clip(jnp.floor(cy).astype(jnp.int32) - 4, 0, H - WIN)
    x0 = jnp.clip(jnp.floor(cx).astype(jnp.int32) - 4, 0, W - WIN)
    packed = z0 * (H * W) + y0 * W + x0
    f_ref[10:11, :] = lax.bitcast_convert_type(packed, jnp.float32)
    f_ref[11:12, :] = y0.astype(jnp.float32)
    f_ref[12:13, :] = x0.astype(jnp.float32)
    zero = cz * 0.0
    f_ref[13:14, :] = zero
    f_ref[14:15, :] = zero
    f_ref[15:16, :] = zero
    i_ref[0:1, :] = z0


def _sc_body(params_hbm, z0_hbm, out_hbm, vol_v, zv, wl, pstage, sem0, sem1):
    sems = (sem0, sem1)
    i32 = jnp.int32
    f32 = jnp.float32
    wid = lax.axis_index("s") * 2 + lax.axis_index("c")
    lo = wid * SLABZ

    yoffs, xoffs, idxcs, padms = [], [], [], []
    for v in range(NVREG):
        p = lax.iota(i32, 16) + (16 * v)
        j = p // 10
        l = p % 10
        padm = p < 100
        yoffs.append(j.astype(f32))
        xoffs.append(l.astype(f32))
        idxcs.append(jnp.where(padm, j * W + l, 0))
        padms.append(padm)

    zero16 = jnp.zeros((16,), f32)

    def zbody(i, c):
        vol_v[pl.ds(i * 16, 16)] = zero16
        return c

    lax.fori_loop(0, SLABW // 16, zbody, 0)

    pltpu.sync_copy(z0_hbm, zv)

    def rbody(i, cnt):
        z0v = zv[pl.ds(i * 16, 16)]
        m = (z0v >= lo - (WIN - 1)) & (z0v <= lo + (SLABZ - 1))
        cs = plsc.cumsum(m.astype(i32))
        posv = cs + (cnt - 1)
        plsc.store_scatter(wl, [posv], lax.iota(i32, 16) + i * 16, mask=m)
        return cnt + jnp.max(cs)

    cnt = lax.fori_loop(0, N // 16, rbody, 0)
    plsc.store_scatter(wl, [lax.iota(i32, 16) + cnt], jnp.full((16,), N, i32))
    nch = (cnt + 15) // 16

    def dma(ci, b):
        gidv = wl[pl.ds(ci * 16, 16)]
        return pltpu.make_async_copy(params_hbm.at[gidv], pstage.at[b],
                                     sems[b])

    @pl.when(nch > 0)
    def _():
        dma(0, 0).start()

    def cpair(cp, c):
        for b in range(2):
            ci = cp * 2 + b

            @pl.when(ci < nch)
            def _():
                @pl.when(ci + 1 < nch)
                def _():
                    dma(ci + 1, 1 - b).start()
                dma(ci, b).wait()

                def gbody(g, c2):
                    row = pstage[b, g, :]

                    def sp(k):
                        return row.at[jnp.full((16,), k, i32)].get(
                            mode="promise_in_bounds")

                    caa, cbb, ccc = sp(0), sp(1), sp(2)
                    cab, cac, cbc = sp(3), sp(4), sp(5)
                    czc, cyc, cxc = sp(6), sp(7), sp(8)
                    dens = sp(9)
                    pks = jnp.max(plsc.bitcast(sp(10), i32))
                    z0s = pks >> 14
                    fb = pks & ((1 << 14) - 1)
                    ybase = sp(11) - cyc
                    xbase = sp(12) - cxc
                    zlo = jnp.maximum(z0s, lo)
                    zhi = jnp.minimum(z0s + WIN, lo + SLABZ)

                    dyv, dxv, pre, idxg = [], [], [], []
                    for v in range(NVREG):
                        dy = ybase + yoffs[v]
                        dx = xbase + xoffs[v]
                        dyv.append(dy)
                        dxv.append(dx)
                        pre.append(cbb * dy * dy + ccc * dx * dx
                                   + cbc * dy * dx)
                        idxg.append(idxcs[v] + fb)

                    def pbody(z, c3):
                        dz = jnp.broadcast_to(z, (16,)).astype(f32) - czc
                        zq = caa * dz * dz
                        zy = cab * dz
                        zx = cac * dz
                        pbz = (z - lo) * (H * W)
                        for v in range(NVREG):
                            m = (zq + pre[v]) + zy * dyv[v] + zx * dxv[v]
                            wv = jnp.exp(-0.5 * m) * dens
                            wv = jnp.where(m <= 9.0, wv, 0.0)
                            plsc.addupdate_scatter(
                                vol_v, [idxg[v] + pbz], wv, mask=padms[v])
                        return c3

                    lax.fori_loop(zlo, zhi, pbody, 0)
                    return c2

                lax.fori_loop(0, 16, gbody, 0)
        return c

    lax.fori_loop(0, (nch + 1) // 2, cpair, 0)

    pltpu.sync_copy(vol_v, out_hbm.at[pl.ds(wid * SLABW, SLABW)])


@jax.jit
def kernel(centers, quaternions, scales, density):
    cen_t = centers.T.reshape(3, N)
    quat_t = quaternions.T.reshape(4, N)
    sc_t = scales.T.reshape(3, N)
    den_t = density.reshape(1, N)

    fparams, iparams = pl.pallas_call(
        _prep_body,
        out_shape=[
            jax.ShapeDtypeStruct((16, N), jnp.float32),
            jax.ShapeDtypeStruct((1, N), jnp.int32),
        ],
    )(cen_t, quat_t, sc_t, den_t)

    params_nt = jnp.concatenate(
        [fparams.T, jnp.zeros((16, 16), jnp.float32)], axis=0)
    z0r = iparams.reshape(N)

    mesh = plsc.VectorSubcoreMesh(core_axis_name="c", subcore_axis_name="s")
    volume_flat = pl.kernel(
        _sc_body,
        out_type=jax.ShapeDtypeStruct((D * H * W,), jnp.float32),
        mesh=mesh,
        scratch_types=[
            pltpu.VMEM((SLABW,), jnp.float32),
            pltpu.VMEM((N,), jnp.int32),
            pltpu.VMEM((N + 16,), jnp.int32),
            pltpu.VMEM((2, 16, 16), jnp.float32),
            pltpu.SemaphoreType.DMA,
            pltpu.SemaphoreType.DMA,
        ],
        compiler_params=pltpu.CompilerParams(
            needs_layout_passes=False, use_tc_tiling_on_sc=False),
    )(params_nt, z0r)
    return volume_flat.reshape(D, H, W)
